# Initial kernel scaffold; baseline (speedup 1.0000x reference)
import jax
import jax.numpy as jnp
from jax.experimental import pallas as pl


def kernel(x, edge_index, batch, target_embedding, params):
    raise NotImplementedError("write your pallas kernel here")


# speedup vs baseline: 5.7790x; 5.7790x over previous
import functools

import jax
import jax.numpy as jnp
from jax import lax
from jax.experimental import pallas as pl
from jax.experimental.pallas import tpu as pltpu
from jax.experimental.pallas import tpu_sc as plsc

_N = 50000
_E = 800000
_B = 128
_D = 32

_NC = 2
_NS = 16
_NW = _NC * _NS
_CH = 128
_G = 14
_NG = 14
_KCH = _G * _NG
_EPT = _CH * _KCH
_EPAD = _EPT * _NW
_NPAD = 50176
_ZR = 392
_ZCH = 3136


def _sc_edge_agg(q, src, dst, zeros):
    mesh = plsc.VectorSubcoreMesh(core_axis_name="c", subcore_axis_name="s")

    @functools.partial(
        pl.kernel,
        out_type=jax.ShapeDtypeStruct((_NC, _NPAD, _D), jnp.float32),
        mesh=mesh,
        compiler_params=pltpu.CompilerParams(use_tc_tiling_on_sc=False),
        scratch_types=[
            pltpu.VMEM((_G, _CH), jnp.int32),
            pltpu.VMEM((_G, _CH), jnp.int32),
            pltpu.VMEM((_CH, _D), jnp.float32),
            pltpu.VMEM((_ZR, _D), jnp.float32),
            pltpu.VMEM_SHARED((_NPAD, _D), jnp.float32),
            pltpu.SemaphoreType.DMA,
        ],
    )
    def k(q_hbm, src_hbm, dst_hbm, z_hbm, out_hbm,
          src_v, dst_v, rows_v, stage_v, acc_sh, sem):
        c = lax.axis_index("c")
        s = lax.axis_index("s")
        wid = c * _NS + s

        pltpu.sync_copy(z_hbm, stage_v)
        zbase = s * _ZCH

        def zbody(j, carry):
            pltpu.sync_copy(stage_v, acc_sh.at[pl.ds(zbase + j * _ZR, _ZR)])
            return carry

        lax.fori_loop(0, _ZCH // _ZR, zbody, 0)
        plsc.subcore_barrier()

        def gbody(g, carry):
            pltpu.sync_copy(src_hbm.at[wid, pl.ds(g * _G, _G)], src_v)
            pltpu.sync_copy(dst_hbm.at[wid, pl.ds(g * _G, _G)], dst_v)

            def ebody(j, c2):
                pltpu.async_copy(q_hbm.at[src_v.at[j]], rows_v, sem).wait()
                pltpu.sync_copy(rows_v, acc_sh.at[dst_v.at[j]], add=True)
                return c2

            lax.fori_loop(0, _G, ebody, 0)
            return carry

        lax.fori_loop(0, _NG, gbody, 0)
        plsc.subcore_barrier()

        def wbody(j, carry):
            off = zbase + j * _ZR
            pltpu.sync_copy(acc_sh.at[pl.ds(off, _ZR)], stage_v)
            pltpu.sync_copy(stage_v, out_hbm.at[c, pl.ds(off, _ZR)])
            return carry

        lax.fori_loop(0, _ZCH // _ZR, wbody, 0)

    return k(q, src, dst, zeros)


_DX = 16
_NX = 5


def _sc_edge_agg0(xchunks, src, dst, zeros):
    mesh = plsc.VectorSubcoreMesh(core_axis_name="c", subcore_axis_name="s")

    @functools.partial(
        pl.kernel,
        out_type=jax.ShapeDtypeStruct((_NC, _NPAD, _NX * _DX), jnp.float32),
        mesh=mesh,
        compiler_params=pltpu.CompilerParams(use_tc_tiling_on_sc=False),
        scratch_types=[
            pltpu.VMEM((_G, _CH), jnp.int32),
            pltpu.VMEM((_G, _CH), jnp.int32),
            pltpu.VMEM((_CH, _DX), jnp.float32),
            pltpu.VMEM((_ZR, _DX), jnp.float32),
            pltpu.VMEM_SHARED((_NPAD, _DX), jnp.float32),
            pltpu.SemaphoreType.DMA,
        ],
    )
    def k(x0_hbm, x1_hbm, x2_hbm, x3_hbm, x4_hbm, src_hbm, dst_hbm, z_hbm,
          out_hbm, src_v, dst_v, rows_v, stage_v, acc_sh, sem):
        c = lax.axis_index("c")
        s = lax.axis_index("s")
        wid = c * _NS + s
        zbase = s * _ZCH

        for kchunk, x_hbm in enumerate((x0_hbm, x1_hbm, x2_hbm, x3_hbm, x4_hbm)):
            pltpu.sync_copy(z_hbm, stage_v)

            def zbody(j, carry):
                pltpu.sync_copy(stage_v, acc_sh.at[pl.ds(zbase + j * _ZR, _ZR)])
                return carry

            lax.fori_loop(0, _ZCH // _ZR, zbody, 0)
            plsc.subcore_barrier()

            def gbody(g, carry):
                pltpu.sync_copy(src_hbm.at[wid, pl.ds(g * _G, _G)], src_v)
                pltpu.sync_copy(dst_hbm.at[wid, pl.ds(g * _G, _G)], dst_v)

                def ebody(j, c2):
                    pltpu.async_copy(x_hbm.at[src_v.at[j]], rows_v, sem).wait()
                    pltpu.sync_copy(rows_v, acc_sh.at[dst_v.at[j]], add=True)
                    return c2

                lax.fori_loop(0, _G, ebody, 0)
                return carry

            lax.fori_loop(0, _NG, gbody, 0)
            plsc.subcore_barrier()

            def wbody(j, carry):
                off = zbase + j * _ZR
                pltpu.sync_copy(acc_sh.at[pl.ds(off, _ZR)], stage_v)
                pltpu.sync_copy(stage_v,
                                out_hbm.at[c, pl.ds(off, _ZR),
                                           pl.ds(kchunk * _DX, _DX)])
                return carry

            lax.fori_loop(0, _ZCH // _ZR, wbody, 0)
            plsc.subcore_barrier()

    return k(*xchunks, src, dst, zeros)


_Bb = 2000


_HI = jax.lax.Precision.HIGHEST


def _stats_accum(v, st_ref):
    s = jnp.sum(v, axis=0, keepdims=True)
    ss = jnp.sum(v * v, axis=0, keepdims=True)

    @pl.when(pl.program_id(0) == 0)
    def _():
        st_ref[...] = jnp.zeros_like(st_ref)

    st_ref[...] += jnp.concatenate([s, ss], axis=0)


def _layer_a_body(h_ref, agg_ref, w1_ref, b1_ref, w2_ref, b2_ref, v_ref, st_ref):
    t = h_ref[...] + (agg_ref[0] + agg_ref[1])[:, :h_ref.shape[1]]
    u = jnp.maximum(jnp.dot(t, w1_ref[...], preferred_element_type=jnp.float32)
                    + b1_ref[...], 0.0)
    v = jnp.dot(u, w2_ref[...], preferred_element_type=jnp.float32)
    v = jnp.maximum(v + b2_ref[...], 0.0)
    v_ref[...] = v
    _stats_accum(v, st_ref)


def _layer_a(h, agg, w1, b1, w2, b2):
    din = h.shape[1]
    da = agg.shape[2]
    return pl.pallas_call(
        _layer_a_body,
        grid=(_N // _Bb,),
        in_specs=[pl.BlockSpec((_Bb, din), lambda i: (i, 0)),
                  pl.BlockSpec((_NC, _Bb, da), lambda i: (0, i, 0)),
                  pl.BlockSpec((din, _D), lambda i: (0, 0)),
                  pl.BlockSpec((1, _D), lambda i: (0, 0)),
                  pl.BlockSpec((_D, _D), lambda i: (0, 0)),
                  pl.BlockSpec((1, _D), lambda i: (0, 0))],
        out_specs=[pl.BlockSpec((_Bb, _D), lambda i: (i, 0)),
                   pl.BlockSpec((2, _D), lambda i: (0, 0))],
        out_shape=[jax.ShapeDtypeStruct((_N, _D), jnp.float32),
                   jax.ShapeDtypeStruct((2, _D), jnp.float32)],
    )(h, agg, w1, b1.reshape(1, _D), w2, b2.reshape(1, _D))


def _bn_affine(st_ref, g_ref, be_ref):
    m = st_ref[0:1, :] * (1.0 / _N)
    var = st_ref[1:2, :] * (1.0 / _N) - m * m
    inv = lax.rsqrt(var + 1e-5)
    scale = inv * g_ref[...]
    shift = be_ref[...] - m * scale
    return scale, shift


def _layer_b_body(v_ref, st_ref, g_ref, be_ref, o_ref):
    scale, shift = _bn_affine(st_ref, g_ref, be_ref)
    o_ref[...] = v_ref[...] * scale + shift


def _layer_b(v, st, g, be):
    return pl.pallas_call(
        _layer_b_body,
        grid=(_N // _Bb,),
        in_specs=[pl.BlockSpec((_Bb, _D), lambda i: (i, 0)),
                  pl.BlockSpec((2, _D), lambda i: (0, 0)),
                  pl.BlockSpec((1, _D), lambda i: (0, 0)),
                  pl.BlockSpec((1, _D), lambda i: (0, 0))],
        out_specs=pl.BlockSpec((_Bb, _D), lambda i: (i, 0)),
        out_shape=jax.ShapeDtypeStruct((_N, _D), jnp.float32),
    )(v, st, g.reshape(1, _D), be.reshape(1, _D))


def _pool_body(v_ref, st_ref, g_ref, be_ref, b_ref, o_ref):
    scale, shift = _bn_affine(st_ref, g_ref, be_ref)
    h = v_ref[...] * scale + shift
    ids = b_ref[...]
    onehot = (ids == lax.broadcasted_iota(jnp.int32, (1, _B), 1)
              ).astype(jnp.float32)
    p = lax.dot_general(onehot, h, (((0,), (0,)), ((), ())), precision=_HI,
                        preferred_element_type=jnp.float32)

    @pl.when(pl.program_id(0) == 0)
    def _():
        o_ref[...] = jnp.zeros_like(o_ref)

    o_ref[...] += p


def _pool(v, st, g, be, batch2d):
    return pl.pallas_call(
        _pool_body,
        grid=(_N // _Bb,),
        in_specs=[pl.BlockSpec((_Bb, _D), lambda i: (i, 0)),
                  pl.BlockSpec((2, _D), lambda i: (0, 0)),
                  pl.BlockSpec((1, _D), lambda i: (0, 0)),
                  pl.BlockSpec((1, _D), lambda i: (0, 0)),
                  pl.BlockSpec((_Bb, 1), lambda i: (i, 0))],
        out_specs=pl.BlockSpec((_B, _D), lambda i: (0, 0)),
        out_shape=jax.ShapeDtypeStruct((_B, _D), jnp.float32),
    )(v, st, g.reshape(1, _D), be.reshape(1, _D), batch2d)


def _head_body(p_ref, t_ref,
               fcxd_w, fcxd_b,
               xt0_w, xt0_b, bn0_g, bn0_b,
               xt1_w, xt1_b, bn1_g, bn1_b,
               xt2_w, xt2_b, bn2_g, bn2_b,
               fc1_w, fc1_b, fc2_w, fc2_b, out_w, out_b,
               o_ref):
    def dmm(a, w_ref, b_ref):
        return jnp.dot(a, w_ref[...],
                       preferred_element_type=jnp.float32) + b_ref[...]

    def bn(a, g_ref, b_ref):
        m = jnp.mean(a, axis=0, keepdims=True)
        var = jnp.mean((a - m) * (a - m), axis=0, keepdims=True)
        return (a - m) / jnp.sqrt(var + 1e-5) * g_ref[...] + b_ref[...]

    xg = jnp.maximum(dmm(p_ref[...], fcxd_w, fcxd_b), 0.0)
    xt = t_ref[...]
    for w_r, b_r, g_r, be_r in ((xt0_w, xt0_b, bn0_g, bn0_b),
                                (xt1_w, xt1_b, bn1_g, bn1_b),
                                (xt2_w, xt2_b, bn2_g, bn2_b)):
        xt = jnp.maximum(bn(dmm(xt, w_r, b_r), g_r, be_r), 0.0)
    xc = jnp.concatenate([xg, xt], axis=1)
    xc = jnp.maximum(dmm(xc, fc1_w, fc1_b), 0.0)
    xc = jnp.maximum(dmm(xc, fc2_w, fc2_b), 0.0)
    o_ref[...] = dmm(xc, out_w, out_b)


def _head(pooled, target, pr):
    args = [pooled, target, pr['fcxd_W'], pr['fcxd_b'].reshape(1, -1)]
    for l in range(3):
        args += [pr[f'xt{l}_W'], pr[f'xt{l}_b'].reshape(1, -1),
                 pr[f'bnxt{l}_g'].reshape(1, -1), pr[f'bnxt{l}_b'].reshape(1, -1)]
    args += [pr['fc1_W'], pr['fc1_b'].reshape(1, -1),
             pr['fc2_W'], pr['fc2_b'].reshape(1, -1),
             pr['out_W'], pr['out_b'].reshape(1, -1)]
    return pl.pallas_call(
        _head_body,
        out_shape=jax.ShapeDtypeStruct((_B, 1), jnp.float32),
    )(*args)


def kernel(x, edge_index, batch, target_embedding, params):
    src = edge_index[0].astype(jnp.int32)
    dst = edge_index[1].astype(jnp.int32)
    pad = _EPAD - _E
    src_p = jnp.concatenate([src, jnp.zeros((pad,), jnp.int32)]
                            ).reshape(_NW, _KCH, _CH)
    dst_p = jnp.concatenate([dst, jnp.full((pad,), _N, jnp.int32)]
                            ).reshape(_NW, _KCH, _CH)
    zeros = jnp.zeros((_ZR, _D), jnp.float32)
    zeros16 = jnp.zeros((_ZR, _DX), jnp.float32)

    xp = jnp.pad(x, ((0, 0), (0, _NX * _DX - x.shape[1])))
    xchunks = [xp[:, k * _DX:(k + 1) * _DX] for k in range(_NX)]

    h = x
    pooled = None
    for l in range(5):
        if l == 0:
            agg = _sc_edge_agg0(xchunks, src_p, dst_p, zeros16)
        else:
            agg = _sc_edge_agg(h, src_p, dst_p, zeros)
        v, st = _layer_a(h, agg, params[f'c{l}_W1'], params[f'c{l}_b1'],
                         params[f'c{l}_W2'], params[f'c{l}_b2'])
        if l < 4:
            h = _layer_b(v, st, params[f'bn{l}_g'], params[f'bn{l}_b'])
        else:
            pooled = _pool(v, st, params[f'bn{l}_g'], params[f'bn{l}_b'],
                           batch.astype(jnp.int32).reshape(_N, 1))
    return _head(pooled, target_embedding, params)
